# baseline (speedup 1.0000x reference)
import functools

import jax
import jax.numpy as jnp
from jax import lax
from jax.experimental import pallas as pl
from jax.experimental.pallas import tpu as pltpu
from jax.experimental.pallas import tpu_sc as plsc

_B, _DIN, _DH, _DZ, _K = 16384, 64, 128, 16, 1024
_BETA = 0.25
_BLK = 256
_NBLK = _B // _BLK
_CK = 128
_NCK = _K // _CK

_PREC = lax.Precision.DEFAULT


def _backbone_body(feats_ref, w1_ref, b1_ref, w2_ref, b2_ref, wmu_ref,
                   bmu_ref, wlv_ref, blv_ref, eps_ref,
                   z_ref, mu_ref, lv_ref):
    f = feats_ref[...]
    h = jnp.maximum(
        lax.dot_general(f, w1_ref[...], (((1,), (0,)), ((), ())),
                        precision=_PREC, preferred_element_type=jnp.float32)
        + b1_ref[...], 0.0)
    h = jnp.maximum(
        lax.dot_general(h, w2_ref[...], (((1,), (0,)), ((), ())),
                        precision=_PREC, preferred_element_type=jnp.float32)
        + b2_ref[...], 0.0)
    mu = lax.dot_general(h, wmu_ref[...], (((1,), (0,)), ((), ())),
                         precision=_PREC,
                         preferred_element_type=jnp.float32) + bmu_ref[...]
    lv = lax.dot_general(h, wlv_ref[...], (((1,), (0,)), ((), ())),
                         precision=_PREC,
                         preferred_element_type=jnp.float32) + blv_ref[...]
    std = jnp.exp(0.5 * lv)
    z = mu + eps_ref[...] * std
    mu_ref[...] = mu
    lv_ref[...] = lv
    z_ref[...] = z


def _argmin_body(z_ref, zsq_ref, csq_ref, cb_ref, idx_ref):
    z = z_ref[...]
    zsq = zsq_ref[...]
    m = jnp.full((_BLK, 1), jnp.inf, jnp.float32)
    best = jnp.zeros((_BLK, 1), jnp.int32)
    for ko in range(_NCK):
        cbc = cb_ref[pl.ds(ko * _CK, _CK), :]
        p = lax.dot_general(z, cbc, (((1,), (1,)), ((), ())),
                            precision=_PREC,
                            preferred_element_type=jnp.float32)
        d = (zsq - 2.0 * p) + csq_ref[:, pl.ds(ko * _CK, _CK)]
        mc = jnp.min(d, axis=1, keepdims=True)
        ii = lax.broadcasted_iota(jnp.int32, d.shape, 1) + (ko * _CK)
        cand = jnp.min(jnp.where(d == mc, ii, _K), axis=1, keepdims=True)
        take = mc < m
        best = jnp.where(take, cand, best)
        m = jnp.minimum(m, mc)
    idx_ref[...] = best.reshape(1, _BLK, 1)


_backbone_call = pl.pallas_call(
    _backbone_body,
    grid=(_NBLK,),
    in_specs=[
        pl.BlockSpec((_BLK, _DIN), lambda i: (i, 0)),
        pl.BlockSpec((_DIN, _DH), lambda i: (0, 0)),
        pl.BlockSpec((1, _DH), lambda i: (0, 0)),
        pl.BlockSpec((_DH, _DH), lambda i: (0, 0)),
        pl.BlockSpec((1, _DH), lambda i: (0, 0)),
        pl.BlockSpec((_DH, _DZ), lambda i: (0, 0)),
        pl.BlockSpec((1, _DZ), lambda i: (0, 0)),
        pl.BlockSpec((_DH, _DZ), lambda i: (0, 0)),
        pl.BlockSpec((1, _DZ), lambda i: (0, 0)),
        pl.BlockSpec((_BLK, _DZ), lambda i: (i, 0)),
    ],
    out_specs=[
        pl.BlockSpec((_BLK, _DZ), lambda i: (i, 0)),
        pl.BlockSpec((_BLK, _DZ), lambda i: (i, 0)),
        pl.BlockSpec((_BLK, _DZ), lambda i: (i, 0)),
    ],
    out_shape=[
        jax.ShapeDtypeStruct((_B, _DZ), jnp.float32),
        jax.ShapeDtypeStruct((_B, _DZ), jnp.float32),
        jax.ShapeDtypeStruct((_B, _DZ), jnp.float32),
    ],
)

_argmin_call = pl.pallas_call(
    _argmin_body,
    grid=(_NBLK,),
    in_specs=[
        pl.BlockSpec((_BLK, _DZ), lambda i: (i, 0)),
        pl.BlockSpec((_BLK, 1), lambda i: (i, 0)),
        pl.BlockSpec((1, _K), lambda i: (0, 0)),
        pl.BlockSpec((_K, _DZ), lambda i: (0, 0)),
    ],
    out_specs=[
        pl.BlockSpec((1, _BLK, 1), lambda i: (i, 0, 0)),
    ],
    out_shape=[
        jax.ShapeDtypeStruct((_NBLK, _BLK, 1), jnp.int32),
    ],
)


_NC, _NS = 2, 16
_NW = _NC * _NS
_BPW = _B // _NW

_SC_GATHER = None


def _sc_gather_fn():
    global _SC_GATHER
    if _SC_GATHER is None:
        mesh = plsc.VectorSubcoreMesh(core_axis_name="c",
                                      subcore_axis_name="s")

        @functools.partial(
            pl.kernel,
            mesh=mesh,
            compiler_params=pltpu.CompilerParams(needs_layout_passes=False),
            out_type=jax.ShapeDtypeStruct((_B * _DZ,), jnp.float32),
            scratch_types=[
                pltpu.VMEM((_BPW,), jnp.int32),
                pltpu.VMEM((_BPW * _DZ,), jnp.float32),
                pltpu.VMEM((_K * _DZ,), jnp.float32),
            ],
        )
        def _sc_gather(cb_hbm, idx_hbm, out_hbm, idx_v, rows_v, cb_v):
            wid = lax.axis_index("s") * _NC + lax.axis_index("c")
            base = wid * _BPW
            pltpu.sync_copy(cb_hbm, cb_v)
            pltpu.sync_copy(idx_hbm.at[pl.ds(base, _BPW)], idx_v)
            lane = lax.iota(jnp.int32, 16)

            def chunk(i, carry):
                v = idx_v[pl.ds(i * 16, 16)] * _DZ
                dst = lane * _DZ + i * (16 * _DZ)
                for j in range(_DZ):
                    vals = plsc.load_gather(cb_v, [v + j])
                    plsc.store_scatter(rows_v, [dst + j], vals)
                return carry

            lax.fori_loop(0, _BPW // 16, chunk, 0)
            pltpu.sync_copy(rows_v,
                            out_hbm.at[pl.ds(base * _DZ, _BPW * _DZ)])

        _SC_GATHER = _sc_gather
    return _SC_GATHER


_EPS_CACHE = None


def _eps():
    global _EPS_CACHE
    if _EPS_CACHE is None:
        _EPS_CACHE = jax.jit(
            lambda: jax.random.normal(jax.random.key(1), (_B, _DZ),
                                      dtype=jnp.float32))()
    return _EPS_CACHE


def kernel(feats, W1, b1, W2, b2, Wmu, bmu, Wlv, blv, codebook):
    eps = _eps()
    z_cont, mu, lv = _backbone_call(
        feats, W1, b1.reshape(1, _DH), W2, b2.reshape(1, _DH),
        Wmu, bmu.reshape(1, _DZ), Wlv, blv.reshape(1, _DZ), eps)
    zsq = jnp.sum(z_cont ** 2, axis=1, keepdims=True)
    csq = jnp.sum(codebook ** 2, axis=1)[None, :]
    (idx3,) = _argmin_call(z_cont, zsq, csq, codebook)
    idx = idx3.reshape(_B)
    z_q = _sc_gather_fn()(codebook.reshape(_K * _DZ), idx).reshape(_B, _DZ)
    loss_commit = jnp.mean((lax.stop_gradient(z_q) - z_cont) ** 2)
    loss_code = jnp.mean((z_q - lax.stop_gradient(z_cont)) ** 2)
    z_q_st = z_cont + lax.stop_gradient(z_q - z_cont)
    vq_loss = _BETA * (loss_commit + loss_code)
    return (z_cont, mu, lv, z_q_st, vq_loss)


# speedup vs baseline: 1.4554x; 1.4554x over previous
import functools

import jax
import jax.numpy as jnp
from jax import lax
from jax.experimental import pallas as pl
from jax.experimental.pallas import tpu as pltpu
from jax.experimental.pallas import tpu_sc as plsc

_B, _DIN, _DH, _DZ, _K = 16384, 64, 128, 16, 1024
_BETA = 0.25
_BLKA = 512
_NBLKA = _B // _BLKA
_BLK = 256
_NBLK = _B // _BLK
_CK = 128
_NCK = _K // _CK

_PREC = lax.Precision.DEFAULT


def _backbone_body(feats_ref, w1_ref, b1_ref, w2_ref, b2_ref, wmu_ref,
                   bmu_ref, wlv_ref, blv_ref, eps_ref,
                   z_ref, mu_ref, lv_ref):
    f = feats_ref[...]
    h = jnp.maximum(
        lax.dot_general(f, w1_ref[...], (((1,), (0,)), ((), ())),
                        precision=_PREC, preferred_element_type=jnp.float32)
        + b1_ref[...], 0.0)
    h = jnp.maximum(
        lax.dot_general(h, w2_ref[...], (((1,), (0,)), ((), ())),
                        precision=_PREC, preferred_element_type=jnp.float32)
        + b2_ref[...], 0.0)
    mu = lax.dot_general(h, wmu_ref[...], (((1,), (0,)), ((), ())),
                         precision=_PREC,
                         preferred_element_type=jnp.float32) + bmu_ref[...]
    lv = lax.dot_general(h, wlv_ref[...], (((1,), (0,)), ((), ())),
                         precision=_PREC,
                         preferred_element_type=jnp.float32) + blv_ref[...]
    std = jnp.exp(0.5 * lv)
    z = mu + eps_ref[...] * std
    mu_ref[...] = mu
    lv_ref[...] = lv
    z_ref[...] = z


def _argmin_body(z_ref, zsqt_ref, csqt_ref, cb2_ref, idx_ref):
    z = z_ref[...]
    zsqt = zsqt_ref[...]
    m = jnp.full((1, _BLK), jnp.inf, jnp.float32)
    best = jnp.zeros((1, _BLK), jnp.int32)
    for ko in range(_NCK):
        cb2c = cb2_ref[pl.ds(ko * _CK, _CK), :]
        p2 = lax.dot_general(cb2c, z, (((1,), (1,)), ((), ())),
                             precision=_PREC,
                             preferred_element_type=jnp.float32)
        d = (zsqt - p2) + csqt_ref[pl.ds(ko * _CK, _CK), :]
        mc = jnp.min(d, axis=0, keepdims=True)
        ii = lax.broadcasted_iota(jnp.int32, d.shape, 0) + (ko * _CK)
        cand = jnp.min(jnp.where(d == mc, ii, _K), axis=0, keepdims=True)
        take = mc < m
        best = jnp.where(take, cand, best)
        m = jnp.minimum(m, mc)
    idx_ref[...] = best.reshape(1, 1, _BLK)


_backbone_call = pl.pallas_call(
    _backbone_body,
    grid=(_NBLKA,),
    in_specs=[
        pl.BlockSpec((_BLKA, _DIN), lambda i: (i, 0)),
        pl.BlockSpec((_DIN, _DH), lambda i: (0, 0)),
        pl.BlockSpec((1, _DH), lambda i: (0, 0)),
        pl.BlockSpec((_DH, _DH), lambda i: (0, 0)),
        pl.BlockSpec((1, _DH), lambda i: (0, 0)),
        pl.BlockSpec((_DH, _DZ), lambda i: (0, 0)),
        pl.BlockSpec((1, _DZ), lambda i: (0, 0)),
        pl.BlockSpec((_DH, _DZ), lambda i: (0, 0)),
        pl.BlockSpec((1, _DZ), lambda i: (0, 0)),
        pl.BlockSpec((_BLKA, _DZ), lambda i: (i, 0)),
    ],
    out_specs=[
        pl.BlockSpec((_BLKA, _DZ), lambda i: (i, 0)),
        pl.BlockSpec((_BLKA, _DZ), lambda i: (i, 0)),
        pl.BlockSpec((_BLKA, _DZ), lambda i: (i, 0)),
    ],
    out_shape=[
        jax.ShapeDtypeStruct((_B, _DZ), jnp.float32),
        jax.ShapeDtypeStruct((_B, _DZ), jnp.float32),
        jax.ShapeDtypeStruct((_B, _DZ), jnp.float32),
    ],
)

_argmin_call = pl.pallas_call(
    _argmin_body,
    grid=(_NBLK,),
    in_specs=[
        pl.BlockSpec((_BLK, _DZ), lambda i: (i, 0)),
        pl.BlockSpec((1, _BLK), lambda i: (0, i)),
        pl.BlockSpec((_K, 1), lambda i: (0, 0)),
        pl.BlockSpec((_K, _DZ), lambda i: (0, 0)),
    ],
    out_specs=[
        pl.BlockSpec((1, 1, _BLK), lambda i: (i, 0, 0)),
    ],
    out_shape=[
        jax.ShapeDtypeStruct((_NBLK, 1, _BLK), jnp.int32),
    ],
)


_NC, _NS = 2, 16
_NW = _NC * _NS
_BPW = _B // _NW

_SC_GATHER = None


def _sc_gather_fn():
    global _SC_GATHER
    if _SC_GATHER is None:
        mesh = plsc.VectorSubcoreMesh(core_axis_name="c",
                                      subcore_axis_name="s")

        @functools.partial(
            pl.kernel,
            mesh=mesh,
            compiler_params=pltpu.CompilerParams(needs_layout_passes=False),
            out_type=[
                jax.ShapeDtypeStruct((_B * _DZ,), jnp.float32),
                jax.ShapeDtypeStruct((_NW * 16,), jnp.float32),
            ],
            scratch_types=[
                pltpu.VMEM((_BPW,), jnp.int32),
                pltpu.VMEM((_BPW * _DZ,), jnp.float32),
                pltpu.VMEM((_BPW * _DZ,), jnp.float32),
                pltpu.VMEM((_K * _DZ,), jnp.float32),
                pltpu.VMEM((16,), jnp.float32),
            ],
        )
        def _sc_gather(cb_hbm, idx_hbm, z_hbm, out_hbm, loss_hbm,
                       idx_v, z_v, st_v, cb_v, acc_v):
            wid = lax.axis_index("s") * _NC + lax.axis_index("c")
            base = wid * _BPW
            pltpu.sync_copy(cb_hbm, cb_v)
            pltpu.sync_copy(idx_hbm.at[pl.ds(base, _BPW)], idx_v)
            pltpu.sync_copy(z_hbm.at[pl.ds(base * _DZ, _BPW * _DZ)], z_v)
            lane = lax.iota(jnp.int32, 16)

            def row(r, acc):
                iv = plsc.load_gather(idx_v, [jnp.full((16,), r, jnp.int32)])
                zq = plsc.load_gather(cb_v, [iv * _DZ + lane])
                zt = z_v[pl.ds(r * _DZ, _DZ)]
                dlt = zq - zt
                st_v[pl.ds(r * _DZ, _DZ)] = zt + dlt
                return acc + dlt * dlt

            acc = lax.fori_loop(0, _BPW, row, jnp.zeros((16,), jnp.float32))
            acc_v[...] = acc
            pltpu.sync_copy(st_v, out_hbm.at[pl.ds(base * _DZ, _BPW * _DZ)])
            pltpu.sync_copy(acc_v, loss_hbm.at[pl.ds(wid * 16, 16)])

        _SC_GATHER = _sc_gather
    return _SC_GATHER


_EPS_CACHE = None


def _eps():
    global _EPS_CACHE
    if _EPS_CACHE is None:
        _EPS_CACHE = jax.jit(
            lambda: jax.random.normal(jax.random.key(1), (_B, _DZ),
                                      dtype=jnp.float32))()
    return _EPS_CACHE


def kernel(feats, W1, b1, W2, b2, Wmu, bmu, Wlv, blv, codebook):
    eps = _eps()
    z_cont, mu, lv = _backbone_call(
        feats, W1, b1.reshape(1, _DH), W2, b2.reshape(1, _DH),
        Wmu, bmu.reshape(1, _DZ), Wlv, blv.reshape(1, _DZ), eps)
    zsqt = jnp.sum(z_cont ** 2, axis=1, keepdims=True).reshape(1, _B)
    csqt = jnp.sum(codebook ** 2, axis=1)[:, None]
    cb2 = codebook * 2.0
    (idx3,) = _argmin_call(z_cont, zsqt, csqt, cb2)
    idx = idx3.reshape(_B)
    z_q_st_flat, losses = _sc_gather_fn()(
        codebook.reshape(_K * _DZ), idx, z_cont.reshape(_B * _DZ))
    z_q_st = z_q_st_flat.reshape(_B, _DZ)
    s = jnp.sum(losses)
    mean_sq = s / (_B * _DZ)
    vq_loss = _BETA * (mean_sq + mean_sq)
    return (z_cont, mu, lv, z_q_st, vq_loss)


# speedup vs baseline: 1.9273x; 1.3242x over previous
import functools

import jax
import jax.numpy as jnp
from jax import lax
from jax.experimental import pallas as pl
from jax.experimental.pallas import tpu as pltpu
from jax.experimental.pallas import tpu_sc as plsc

_B, _DIN, _DH, _DZ, _K = 16384, 64, 128, 16, 1024
_BETA = 0.25
_BLKA = 512
_NBLKA = _B // _BLKA
_BLK = 256
_NBLK = _B // _BLK
_CK = 128
_NCK = _K // _CK

_PREC = lax.Precision.DEFAULT


def _backbone_body(feats_ref, w1_ref, b1_ref, w2_ref, b2_ref, wmu_ref,
                   bmu_ref, wlv_ref, blv_ref, eps_ref,
                   z_ref, mu_ref, lv_ref):
    f = feats_ref[...]
    h = jnp.maximum(
        lax.dot_general(f, w1_ref[...], (((1,), (0,)), ((), ())),
                        precision=_PREC, preferred_element_type=jnp.float32)
        + b1_ref[...], 0.0)
    h = jnp.maximum(
        lax.dot_general(h, w2_ref[...], (((1,), (0,)), ((), ())),
                        precision=_PREC, preferred_element_type=jnp.float32)
        + b2_ref[...], 0.0)
    mu = lax.dot_general(h, wmu_ref[...], (((1,), (0,)), ((), ())),
                         precision=_PREC,
                         preferred_element_type=jnp.float32) + bmu_ref[...]
    lv = lax.dot_general(h, wlv_ref[...], (((1,), (0,)), ((), ())),
                         precision=_PREC,
                         preferred_element_type=jnp.float32) + blv_ref[...]
    std = jnp.exp(0.5 * lv)
    z = mu + eps_ref[...] * std
    mu_ref[...] = mu
    lv_ref[...] = lv
    z_ref[...] = z


def _argmin_body(z_ref, zsqt_ref, csqt_ref, cb2_ref, idx_ref):
    z = z_ref[...]
    zsqt = zsqt_ref[...]
    m = jnp.full((1, _BLK), jnp.inf, jnp.float32)
    best = jnp.zeros((1, _BLK), jnp.int32)
    for ko in range(_NCK):
        cb2c = cb2_ref[pl.ds(ko * _CK, _CK), :]
        p2 = lax.dot_general(cb2c, z, (((1,), (1,)), ((), ())),
                             precision=_PREC,
                             preferred_element_type=jnp.float32)
        d = (zsqt - p2) + csqt_ref[pl.ds(ko * _CK, _CK), :]
        mc = jnp.min(d, axis=0, keepdims=True)
        ii = lax.broadcasted_iota(jnp.int32, d.shape, 0) + (ko * _CK)
        cand = jnp.min(jnp.where(d == mc, ii, _K), axis=0, keepdims=True)
        take = mc < m
        best = jnp.where(take, cand, best)
        m = jnp.minimum(m, mc)
    idx_ref[...] = best.reshape(1, 1, _BLK)


_backbone_call = pl.pallas_call(
    _backbone_body,
    grid=(_NBLKA,),
    in_specs=[
        pl.BlockSpec((_BLKA, _DIN), lambda i: (i, 0)),
        pl.BlockSpec((_DIN, _DH), lambda i: (0, 0)),
        pl.BlockSpec((1, _DH), lambda i: (0, 0)),
        pl.BlockSpec((_DH, _DH), lambda i: (0, 0)),
        pl.BlockSpec((1, _DH), lambda i: (0, 0)),
        pl.BlockSpec((_DH, _DZ), lambda i: (0, 0)),
        pl.BlockSpec((1, _DZ), lambda i: (0, 0)),
        pl.BlockSpec((_DH, _DZ), lambda i: (0, 0)),
        pl.BlockSpec((1, _DZ), lambda i: (0, 0)),
        pl.BlockSpec((_BLKA, _DZ), lambda i: (i, 0)),
    ],
    out_specs=[
        pl.BlockSpec((_BLKA, _DZ), lambda i: (i, 0)),
        pl.BlockSpec((_BLKA, _DZ), lambda i: (i, 0)),
        pl.BlockSpec((_BLKA, _DZ), lambda i: (i, 0)),
    ],
    out_shape=[
        jax.ShapeDtypeStruct((_B, _DZ), jnp.float32),
        jax.ShapeDtypeStruct((_B, _DZ), jnp.float32),
        jax.ShapeDtypeStruct((_B, _DZ), jnp.float32),
    ],
)

_argmin_call = pl.pallas_call(
    _argmin_body,
    grid=(_NBLK,),
    in_specs=[
        pl.BlockSpec((_BLK, _DZ), lambda i: (i, 0)),
        pl.BlockSpec((1, _BLK), lambda i: (0, i)),
        pl.BlockSpec((_K, 1), lambda i: (0, 0)),
        pl.BlockSpec((_K, _DZ), lambda i: (0, 0)),
    ],
    out_specs=[
        pl.BlockSpec((1, 1, _BLK), lambda i: (i, 0, 0)),
    ],
    out_shape=[
        jax.ShapeDtypeStruct((_NBLK, 1, _BLK), jnp.int32),
    ],
)


_NC, _NS = 2, 16
_NW = _NC * _NS
_BPW = _B // _NW

_SC_GATHER = None


def _sc_gather_fn():
    global _SC_GATHER
    if _SC_GATHER is None:
        mesh = plsc.VectorSubcoreMesh(core_axis_name="c",
                                      subcore_axis_name="s")

        @functools.partial(
            pl.kernel,
            mesh=mesh,
            compiler_params=pltpu.CompilerParams(needs_layout_passes=False),
            out_type=[
                jax.ShapeDtypeStruct((_B * _DZ,), jnp.float32),
                jax.ShapeDtypeStruct((_NW * 16,), jnp.float32),
            ],
            scratch_types=[
                pltpu.VMEM((_BPW,), jnp.int32),
                pltpu.VMEM((_BPW * _DZ,), jnp.float32),
                pltpu.VMEM((_BPW * _DZ,), jnp.float32),
                pltpu.VMEM((_K * _DZ,), jnp.float32),
                pltpu.VMEM((16,), jnp.float32),
            ],
        )
        def _sc_gather(cb_hbm, idx_hbm, z_hbm, out_hbm, loss_hbm,
                       idx_v, z_v, st_v, cb_v, acc_v):
            wid = lax.axis_index("s") * _NC + lax.axis_index("c")
            base = wid * _BPW
            pltpu.sync_copy(cb_hbm, cb_v)
            pltpu.sync_copy(idx_hbm.at[pl.ds(base, _BPW)], idx_v)
            pltpu.sync_copy(z_hbm.at[pl.ds(base * _DZ, _BPW * _DZ)], z_v)
            lane = lax.iota(jnp.int32, 16)

            def row(r, acc):
                iv = plsc.load_gather(idx_v, [jnp.full((16,), r, jnp.int32)])
                zq = plsc.load_gather(cb_v, [iv * _DZ + lane])
                zt = z_v[pl.ds(r * _DZ, _DZ)]
                dlt = zq - zt
                st_v[pl.ds(r * _DZ, _DZ)] = zt + dlt
                return acc + dlt * dlt

            acc = lax.fori_loop(0, _BPW, row, jnp.zeros((16,), jnp.float32))
            acc_v[...] = acc
            pltpu.sync_copy(st_v, out_hbm.at[pl.ds(base * _DZ, _BPW * _DZ)])
            pltpu.sync_copy(acc_v, loss_hbm.at[pl.ds(wid * 16, 16)])

        _SC_GATHER = _sc_gather
    return _SC_GATHER


_EPS_CACHE = None


def _eps():
    global _EPS_CACHE
    if _EPS_CACHE is None:
        with jax.ensure_compile_time_eval():
            _EPS_CACHE = jax.random.normal(jax.random.key(1), (_B, _DZ),
                                           dtype=jnp.float32)
    return _EPS_CACHE


def kernel(feats, W1, b1, W2, b2, Wmu, bmu, Wlv, blv, codebook):
    eps = _eps()
    z_cont, mu, lv = _backbone_call(
        feats, W1, b1.reshape(1, _DH), W2, b2.reshape(1, _DH),
        Wmu, bmu.reshape(1, _DZ), Wlv, blv.reshape(1, _DZ), eps)
    zsqt = jnp.sum(z_cont ** 2, axis=1, keepdims=True).reshape(1, _B)
    csqt = jnp.sum(codebook ** 2, axis=1)[:, None]
    cb2 = codebook * 2.0
    (idx3,) = _argmin_call(z_cont, zsqt, csqt, cb2)
    idx = idx3.reshape(_B)
    z_q_st_flat, losses = _sc_gather_fn()(
        codebook.reshape(_K * _DZ), idx, z_cont.reshape(_B * _DZ))
    z_q_st = z_q_st_flat.reshape(_B, _DZ)
    s = jnp.sum(losses)
    mean_sq = s / (_B * _DZ)
    vq_loss = _BETA * (mean_sq + mean_sq)
    return (z_cont, mu, lv, z_q_st, vq_loss)


# speedup vs baseline: 2.1709x; 1.1264x over previous
import functools

import jax
import jax.numpy as jnp
from jax import lax
from jax.experimental import pallas as pl
from jax.experimental.pallas import tpu as pltpu
from jax.experimental.pallas import tpu_sc as plsc

_B, _DIN, _DH, _DZ, _K = 16384, 64, 128, 16, 1024
_BETA = 0.25
_BLKA = 512
_NBLKA = _B // _BLKA
_BLK = 256
_NBLK = _B // _BLK
_CK = 128
_NCK = _K // _CK

_PREC = lax.Precision.DEFAULT


def _fused_body(feats_ref, w1_ref, b1_ref, w2_ref, b2_ref, wmu_ref,
                bmu_ref, wlv_ref, blv_ref, eps_ref, csqt_ref, cb2_ref,
                z_ref, mu_ref, lv_ref, idx_ref):
    f = feats_ref[...]
    h = jnp.maximum(
        lax.dot_general(f, w1_ref[...], (((1,), (0,)), ((), ())),
                        precision=_PREC, preferred_element_type=jnp.float32)
        + b1_ref[...], 0.0)
    h = jnp.maximum(
        lax.dot_general(h, w2_ref[...], (((1,), (0,)), ((), ())),
                        precision=_PREC, preferred_element_type=jnp.float32)
        + b2_ref[...], 0.0)
    mu = lax.dot_general(h, wmu_ref[...], (((1,), (0,)), ((), ())),
                         precision=_PREC,
                         preferred_element_type=jnp.float32) + bmu_ref[...]
    lv = lax.dot_general(h, wlv_ref[...], (((1,), (0,)), ((), ())),
                         precision=_PREC,
                         preferred_element_type=jnp.float32) + blv_ref[...]
    std = jnp.exp(0.5 * lv)
    z = mu + eps_ref[...] * std
    mu_ref[...] = mu
    lv_ref[...] = lv
    z_ref[...] = z
    zt = z.T
    zt2 = zt * zt
    s = zt2[0:8, :] + zt2[8:16, :]
    s = s[0:4, :] + s[4:8, :]
    s = s[0:2, :] + s[2:4, :]
    zsqt = s[0:1, :] + s[1:2, :]
    m = jnp.full((1, _BLK), jnp.inf, jnp.float32)
    best = jnp.zeros((1, _BLK), jnp.int32)
    iota_loc = lax.broadcasted_iota(jnp.int32, (_CK, _BLK), 0)
    for ko in range(_NCK):
        cb2c = cb2_ref[pl.ds(ko * _CK, _CK), :]
        p2 = lax.dot_general(cb2c, z, (((1,), (1,)), ((), ())),
                             precision=_PREC,
                             preferred_element_type=jnp.float32)
        d = (zsqt - p2) + csqt_ref[pl.ds(ko * _CK, _CK), :]
        mc = jnp.min(d, axis=0, keepdims=True)
        cand = jnp.min(jnp.where(d == mc, iota_loc, _K),
                       axis=0, keepdims=True) + (ko * _CK)
        take = mc < m
        best = jnp.where(take, cand, best)
        m = jnp.minimum(m, mc)
    idx_ref[...] = best.reshape(1, 1, _BLK)


_fused_call = pl.pallas_call(
    _fused_body,
    grid=(_NBLK,),
    in_specs=[
        pl.BlockSpec((_BLK, _DIN), lambda i: (i, 0)),
        pl.BlockSpec((_DIN, _DH), lambda i: (0, 0)),
        pl.BlockSpec((1, _DH), lambda i: (0, 0)),
        pl.BlockSpec((_DH, _DH), lambda i: (0, 0)),
        pl.BlockSpec((1, _DH), lambda i: (0, 0)),
        pl.BlockSpec((_DH, _DZ), lambda i: (0, 0)),
        pl.BlockSpec((1, _DZ), lambda i: (0, 0)),
        pl.BlockSpec((_DH, _DZ), lambda i: (0, 0)),
        pl.BlockSpec((1, _DZ), lambda i: (0, 0)),
        pl.BlockSpec((_BLK, _DZ), lambda i: (i, 0)),
        pl.BlockSpec((_K, 1), lambda i: (0, 0)),
        pl.BlockSpec((_K, _DZ), lambda i: (0, 0)),
    ],
    out_specs=[
        pl.BlockSpec((_BLK, _DZ), lambda i: (i, 0)),
        pl.BlockSpec((_BLK, _DZ), lambda i: (i, 0)),
        pl.BlockSpec((_BLK, _DZ), lambda i: (i, 0)),
        pl.BlockSpec((1, 1, _BLK), lambda i: (i, 0, 0)),
    ],
    out_shape=[
        jax.ShapeDtypeStruct((_B, _DZ), jnp.float32),
        jax.ShapeDtypeStruct((_B, _DZ), jnp.float32),
        jax.ShapeDtypeStruct((_B, _DZ), jnp.float32),
        jax.ShapeDtypeStruct((_NBLK, 1, _BLK), jnp.int32),
    ],
)


_NC, _NS = 2, 16
_NW = _NC * _NS
_BPW = _B // _NW

_SC_GATHER = None


def _sc_gather_fn():
    global _SC_GATHER
    if _SC_GATHER is None:
        mesh = plsc.VectorSubcoreMesh(core_axis_name="c",
                                      subcore_axis_name="s")

        @functools.partial(
            pl.kernel,
            mesh=mesh,
            compiler_params=pltpu.CompilerParams(needs_layout_passes=False),
            out_type=[
                jax.ShapeDtypeStruct((_B * _DZ,), jnp.float32),
                jax.ShapeDtypeStruct((_NW * 16,), jnp.float32),
            ],
            scratch_types=[
                pltpu.VMEM((_BPW,), jnp.int32),
                pltpu.VMEM((_BPW * _DZ,), jnp.float32),
                pltpu.VMEM((_BPW * _DZ,), jnp.float32),
                pltpu.VMEM((_K * _DZ,), jnp.float32),
                pltpu.VMEM((16,), jnp.float32),
            ],
        )
        def _sc_gather(cb_hbm, idx_hbm, z_hbm, out_hbm, loss_hbm,
                       idx_v, z_v, st_v, cb_v, acc_v):
            wid = lax.axis_index("s") * _NC + lax.axis_index("c")
            base = wid * _BPW
            pltpu.sync_copy(cb_hbm, cb_v)
            pltpu.sync_copy(idx_hbm.at[pl.ds(base, _BPW)], idx_v)
            pltpu.sync_copy(z_hbm.at[pl.ds(base * _DZ, _BPW * _DZ)], z_v)
            lane = lax.iota(jnp.int32, 16)

            def row(r, acc):
                iv = plsc.load_gather(idx_v, [jnp.full((16,), r, jnp.int32)])
                zq = plsc.load_gather(cb_v, [iv * _DZ + lane])
                zt = z_v[pl.ds(r * _DZ, _DZ)]
                dlt = zq - zt
                st_v[pl.ds(r * _DZ, _DZ)] = zt + dlt
                return acc + dlt * dlt

            acc = lax.fori_loop(0, _BPW, row, jnp.zeros((16,), jnp.float32))
            acc_v[...] = acc
            pltpu.sync_copy(st_v, out_hbm.at[pl.ds(base * _DZ, _BPW * _DZ)])
            pltpu.sync_copy(acc_v, loss_hbm.at[pl.ds(wid * 16, 16)])

        _SC_GATHER = _sc_gather
    return _SC_GATHER


_EPS_CACHE = None


def _eps():
    global _EPS_CACHE
    if _EPS_CACHE is None:
        with jax.ensure_compile_time_eval():
            _EPS_CACHE = jax.random.normal(jax.random.key(1), (_B, _DZ),
                                           dtype=jnp.float32)
    return _EPS_CACHE


def kernel(feats, W1, b1, W2, b2, Wmu, bmu, Wlv, blv, codebook):
    eps = _eps()
    csqt = jnp.sum(codebook ** 2, axis=1)[:, None]
    cb2 = codebook * 2.0
    z_cont, mu, lv, idx3 = _fused_call(
        feats, W1, b1.reshape(1, _DH), W2, b2.reshape(1, _DH),
        Wmu, bmu.reshape(1, _DZ), Wlv, blv.reshape(1, _DZ), eps, csqt, cb2)
    idx = idx3.reshape(_B)
    z_q_st_flat, losses = _sc_gather_fn()(
        codebook.reshape(_K * _DZ), idx, z_cont.reshape(_B * _DZ))
    z_q_st = z_q_st_flat.reshape(_B, _DZ)
    s = jnp.sum(losses)
    mean_sq = s / (_B * _DZ)
    vq_loss = _BETA * (mean_sq + mean_sq)
    return (z_cont, mu, lv, z_q_st, vq_loss)


# speedup vs baseline: 2.2579x; 1.0401x over previous
import functools

import jax
import jax.numpy as jnp
from jax import lax
from jax.experimental import pallas as pl
from jax.experimental.pallas import tpu as pltpu
from jax.experimental.pallas import tpu_sc as plsc

_B, _DIN, _DH, _DZ, _K = 16384, 64, 128, 16, 1024
_BETA = 0.25
_BLKA = 512
_NBLKA = _B // _BLKA
_BLK = 256
_NBLK = _B // _BLK
_CK = 128
_NCK = _K // _CK

_PREC = lax.Precision.DEFAULT


def _fused_body(feats_ref, w1_ref, b1_ref, w2_ref, b2_ref, wmu_ref,
                bmu_ref, wlv_ref, blv_ref, eps_ref, csqt_ref, cb2_ref,
                z_ref, mu_ref, lv_ref, idx_ref):
    f = feats_ref[...]
    h = jnp.maximum(
        lax.dot_general(f, w1_ref[...], (((1,), (0,)), ((), ())),
                        precision=_PREC, preferred_element_type=jnp.float32)
        + b1_ref[...], 0.0)
    h = jnp.maximum(
        lax.dot_general(h, w2_ref[...], (((1,), (0,)), ((), ())),
                        precision=_PREC, preferred_element_type=jnp.float32)
        + b2_ref[...], 0.0)
    mu = lax.dot_general(h, wmu_ref[...], (((1,), (0,)), ((), ())),
                         precision=_PREC,
                         preferred_element_type=jnp.float32) + bmu_ref[...]
    lv = lax.dot_general(h, wlv_ref[...], (((1,), (0,)), ((), ())),
                         precision=_PREC,
                         preferred_element_type=jnp.float32) + blv_ref[...]
    std = jnp.exp(0.5 * lv)
    z = mu + eps_ref[...] * std
    mu_ref[...] = mu
    lv_ref[...] = lv
    z_ref[...] = z
    zt = z.T
    zt2 = zt * zt
    s = zt2[0:8, :] + zt2[8:16, :]
    s = s[0:4, :] + s[4:8, :]
    s = s[0:2, :] + s[2:4, :]
    zsqt = s[0:1, :] + s[1:2, :]
    m = jnp.full((1, _BLK), jnp.inf, jnp.float32)
    best = jnp.zeros((1, _BLK), jnp.int32)
    iota_loc = lax.broadcasted_iota(jnp.int32, (_CK, _BLK), 0)
    for ko in range(_NCK):
        cb2c = cb2_ref[pl.ds(ko * _CK, _CK), :]
        p2 = lax.dot_general(cb2c, z, (((1,), (1,)), ((), ())),
                             precision=_PREC,
                             preferred_element_type=jnp.float32)
        d = (zsqt - p2) + csqt_ref[pl.ds(ko * _CK, _CK), :]
        mc = jnp.min(d, axis=0, keepdims=True)
        cand = jnp.min(jnp.where(d == mc, iota_loc, _K),
                       axis=0, keepdims=True) + (ko * _CK)
        take = mc < m
        best = jnp.where(take, cand, best)
        m = jnp.minimum(m, mc)
    idx_ref[...] = best.reshape(1, 1, _BLK)


_fused_call = pl.pallas_call(
    _fused_body,
    grid=(_NBLK,),
    in_specs=[
        pl.BlockSpec((_BLK, _DIN), lambda i: (i, 0)),
        pl.BlockSpec((_DIN, _DH), lambda i: (0, 0)),
        pl.BlockSpec((1, _DH), lambda i: (0, 0)),
        pl.BlockSpec((_DH, _DH), lambda i: (0, 0)),
        pl.BlockSpec((1, _DH), lambda i: (0, 0)),
        pl.BlockSpec((_DH, _DZ), lambda i: (0, 0)),
        pl.BlockSpec((1, _DZ), lambda i: (0, 0)),
        pl.BlockSpec((_DH, _DZ), lambda i: (0, 0)),
        pl.BlockSpec((1, _DZ), lambda i: (0, 0)),
        pl.BlockSpec((_BLK, _DZ), lambda i: (i, 0)),
        pl.BlockSpec((_K, 1), lambda i: (0, 0)),
        pl.BlockSpec((_K, _DZ), lambda i: (0, 0)),
    ],
    out_specs=[
        pl.BlockSpec((_BLK, _DZ), lambda i: (i, 0)),
        pl.BlockSpec((_BLK, _DZ), lambda i: (i, 0)),
        pl.BlockSpec((_BLK, _DZ), lambda i: (i, 0)),
        pl.BlockSpec((1, 1, _BLK), lambda i: (i, 0, 0)),
    ],
    out_shape=[
        jax.ShapeDtypeStruct((_B, _DZ), jnp.float32),
        jax.ShapeDtypeStruct((_B, _DZ), jnp.float32),
        jax.ShapeDtypeStruct((_B, _DZ), jnp.float32),
        jax.ShapeDtypeStruct((_NBLK, 1, _BLK), jnp.int32),
    ],
)


_NC, _NS = 2, 16
_NW = _NC * _NS
_BPW = _B // _NW

_SC_GATHER = None


def _sc_gather_fn():
    global _SC_GATHER
    if _SC_GATHER is None:
        mesh = plsc.VectorSubcoreMesh(core_axis_name="c",
                                      subcore_axis_name="s")

        @functools.partial(
            pl.kernel,
            mesh=mesh,
            compiler_params=pltpu.CompilerParams(needs_layout_passes=False),
            out_type=[
                jax.ShapeDtypeStruct((_B, _DZ), jnp.float32),
                jax.ShapeDtypeStruct((_NW * 16,), jnp.float32),
            ],
            scratch_types=[
                pltpu.VMEM((_B // _BLK // _NW, 1, _BLK), jnp.int32),
                pltpu.VMEM((_BPW * _DZ,), jnp.float32),
                pltpu.VMEM((_BPW, _DZ), jnp.float32),
                pltpu.VMEM((_K * _DZ,), jnp.float32),
                pltpu.VMEM((16,), jnp.float32),
            ],
        )
        def _sc_gather(cb_hbm, idx_hbm, z_hbm, out_hbm, loss_hbm,
                       idx_v, z_v, st_v, cb_v, acc_v):
            nblk_w = _B // _BLK // _NW
            wid = lax.axis_index("s") * _NC + lax.axis_index("c")
            base = wid * _BPW
            pltpu.sync_copy(cb_hbm, cb_v)
            pltpu.sync_copy(idx_hbm.at[pl.ds(wid * nblk_w, nblk_w)], idx_v)
            pltpu.sync_copy(z_hbm.at[pl.ds(base * _DZ, _BPW * _DZ)], z_v)
            lane = lax.iota(jnp.int32, 16)
            zeros = jnp.zeros((16,), jnp.int32)

            def row(r, acc):
                iv = plsc.load_gather(
                    idx_v, [jnp.full((16,), r // _BLK, jnp.int32), zeros,
                            jnp.full((16,), r % _BLK, jnp.int32)])
                zq = plsc.load_gather(cb_v, [iv * _DZ + lane])
                zt = z_v[pl.ds(r * _DZ, _DZ)]
                dlt = zq - zt
                st_v[r] = zq
                return acc + dlt * dlt

            acc = lax.fori_loop(0, _BPW, row, jnp.zeros((16,), jnp.float32))
            acc_v[...] = acc
            pltpu.sync_copy(st_v, out_hbm.at[pl.ds(base, _BPW)])
            pltpu.sync_copy(acc_v, loss_hbm.at[pl.ds(wid * 16, 16)])

        _SC_GATHER = _sc_gather
    return _SC_GATHER


_EPS_CACHE = None


def _eps():
    global _EPS_CACHE
    if _EPS_CACHE is None:
        with jax.ensure_compile_time_eval():
            _EPS_CACHE = jax.random.normal(jax.random.key(1), (_B, _DZ),
                                           dtype=jnp.float32)
    return _EPS_CACHE


def kernel(feats, W1, b1, W2, b2, Wmu, bmu, Wlv, blv, codebook):
    eps = _eps()
    csqt = jnp.sum(codebook ** 2, axis=1)[:, None]
    cb2 = codebook * 2.0
    z_cont, mu, lv, idx3 = _fused_call(
        feats, W1, b1.reshape(1, _DH), W2, b2.reshape(1, _DH),
        Wmu, bmu.reshape(1, _DZ), Wlv, blv.reshape(1, _DZ), eps, csqt, cb2)
    z_q_st, losses = _sc_gather_fn()(
        codebook.reshape(_K * _DZ), idx3, z_cont.reshape(_B * _DZ))
    s = jnp.sum(losses)
    mean_sq = s / (_B * _DZ)
    vq_loss = _BETA * (mean_sq + mean_sq)
    return (z_cont, mu, lv, z_q_st, vq_loss)
